# baseline (speedup 1.0000x reference)
import functools

import jax
import jax.numpy as jnp
from jax import lax
from jax.experimental import pallas as pl
from jax.experimental.pallas import tpu as pltpu
from jax.experimental.pallas import tpu_sc as plsc

_SLAB = 15
_CHK = 8
_CPS = 1 << (_SLAB - _CHK)


@functools.lru_cache(maxsize=None)
def _build_embed_kernel(B, V, D):
    info = plsc.get_sparse_core_info()
    nw = info.num_cores * info.num_subcores
    L = info.num_lanes
    n_vecs = B // L
    chunk_rows = 1 << _CHK
    tail_rows = V & (chunk_rows - 1)

    mesh = plsc.VectorSubcoreMesh(core_axis_name="c", subcore_axis_name="s")

    @functools.partial(
        pl.kernel,
        mesh=mesh,
        compiler_params=pltpu.CompilerParams(needs_layout_passes=False),
        out_type=jax.ShapeDtypeStruct((B, 1, D), jnp.float32),
        scratch_types=[
            pltpu.VMEM((B + L,), jnp.int32),
            pltpu.VMEM((B + L,), jnp.int32),
            pltpu.VMEM((B + L,), jnp.int32),
            pltpu.VMEM((2, chunk_rows, D), jnp.float32),
            pltpu.VMEM((L,), jnp.int32),
            pltpu.VMEM((L,), jnp.int32),
            pltpu.SemaphoreType.DMA,
            pltpu.SemaphoreType.DMA,
            pltpu.SemaphoreType.DMA,
        ],
    )
    def embed(idx_hbm, table_hbm, out_hbm, lab_v, mlab_v, mpos_v, buf_v,
              clab_v, cpos_v, sem_a, sem_b, sem_o):
        wid = lax.axis_index("s") * info.num_cores + lax.axis_index("c")
        slab_base = wid * (1 << _SLAB)
        pltpu.sync_copy(idx_hbm, lab_v.at[pl.ds(0, B)])
        wid_vec = jnp.full((L,), wid, jnp.int32)

        def scan_body(i, cnt):
            labs = lab_v[pl.ds(i * L, L)]
            m = lax.shift_right_logical(labs, _SLAB) == wid_vec
            pos = lax.iota(jnp.int32, L) + i * L
            plsc.store_compressed(mlab_v.at[pl.ds(cnt, L)], labs, mask=m)
            plsc.store_compressed(mpos_v.at[pl.ds(cnt, L)], pos, mask=m)
            return cnt + plsc.all_reduce_population_count(m)[0]

        cnt = lax.fori_loop(0, n_vecs, scan_body, 0)
        sentinel = jnp.full((L,), jnp.int32(0x7FFFFFFF), jnp.int32)
        mlab_v[pl.ds(cnt, L)] = sentinel
        n_mvecs = lax.shift_right_logical(cnt + (L - 1), 4)

        def chunk_base(k):
            return slab_base + k * chunk_rows

        def stream_in(k, buf, sem):
            base = chunk_base(k)

            @pl.when(base + chunk_rows <= V)
            def _():
                pltpu.async_copy(
                    table_hbm.at[pl.ds(base, chunk_rows)],
                    buf_v.at[buf], sem,
                )

            if tail_rows:
                @pl.when(jnp.logical_and(base < V, base + chunk_rows > V))
                def _():
                    pltpu.async_copy(
                        table_hbm.at[pl.ds(V - tail_rows, tail_rows)],
                        buf_v.at[buf, pl.ds(0, tail_rows)], sem,
                    )

        def drain_in(k, buf, sem):
            base = chunk_base(k)

            @pl.when(base + chunk_rows <= V)
            def _():
                pltpu.make_async_copy(
                    table_hbm.at[pl.ds(0, chunk_rows)], buf_v.at[buf], sem
                ).wait()

            if tail_rows:
                @pl.when(jnp.logical_and(base < V, base + chunk_rows > V))
                def _():
                    pltpu.make_async_copy(
                        table_hbm.at[pl.ds(0, tail_rows)],
                        buf_v.at[buf, pl.ds(0, tail_rows)], sem,
                    ).wait()

        def drain_out(m):
            def body(j, carry):
                pltpu.make_async_copy(
                    buf_v.at[0, 0], out_hbm.at[0, 0], sem_o
                ).wait()
                return carry

            lax.fori_loop(0, m, body, 0)

        def process(k, buf):
            base = chunk_base(k)
            cid_vec = jnp.full((L,), base // chunk_rows, jnp.int32)

            def sel_body(j, m_out):
                ml = mlab_v[pl.ds(j * L, L)]
                mp = mpos_v[pl.ds(j * L, L)]
                m = lax.shift_right_logical(ml, _CHK) == cid_vec
                clab_v[pl.ds(0, L)] = sentinel
                plsc.store_compressed(clab_v.at[pl.ds(0, L)], ml, mask=m)
                plsc.store_compressed(cpos_v.at[pl.ds(0, L)], mp, mask=m)
                mc = plsc.all_reduce_population_count(m)[0]
                cl = clab_v[pl.ds(0, L)]
                cp = cpos_v[pl.ds(0, L)]
                for t in range(L):
                    @pl.when(t < mc)
                    def _():
                        locrow = lax.bitwise_and(cl[t], chunk_rows - 1)
                        pltpu.async_copy(
                            buf_v.at[buf, locrow], out_hbm.at[cp[t], 0], sem_o
                        )
                return m_out + mc

            return lax.fori_loop(0, n_mvecs, sel_body, 0)

        stream_in(0, 0, sem_a)

        def pair_body(i, m_prev):
            k0 = 2 * i
            drain_out(m_prev)

            @pl.when(k0 + 1 < _CPS)
            def _():
                stream_in(k0 + 1, 1, sem_b)
            drain_in(k0, 0, sem_a)
            m0 = process(k0, 0)

            drain_out(m0)

            @pl.when(k0 + 2 < _CPS)
            def _():
                stream_in(k0 + 2, 0, sem_a)
            drain_in(k0 + 1, 1, sem_b)
            m1 = process(k0 + 1, 1)
            return m1

        m_last = lax.fori_loop(0, _CPS // 2, pair_body, 0)
        drain_out(m_last)

    return embed


def kernel(class_labels, table):
    B = class_labels.shape[0]
    V, D = table.shape
    embed = _build_embed_kernel(B, V, D)
    return embed(class_labels.astype(jnp.int32), table)


# speedup vs baseline: 2.9892x; 2.9892x over previous
import functools

import jax
import jax.numpy as jnp
from jax import lax
from jax.experimental import pallas as pl
from jax.experimental.pallas import tpu as pltpu
from jax.experimental.pallas import tpu_sc as plsc


@functools.lru_cache(maxsize=None)
def _build_embed_kernel(B, V, D):
    info = plsc.get_sparse_core_info()
    nw = info.num_cores * info.num_subcores
    b_per_w = B // nw

    mesh = plsc.VectorSubcoreMesh(core_axis_name="c", subcore_axis_name="s")

    @functools.partial(
        pl.kernel,
        mesh=mesh,
        compiler_params=pltpu.CompilerParams(needs_layout_passes=False),
        out_type=jax.ShapeDtypeStruct((B, D), jnp.float32),
        scratch_types=[
            pltpu.VMEM((b_per_w,), jnp.int32),
            pltpu.VMEM((b_per_w, D), jnp.float32),
            pltpu.SemaphoreType.DMA,
        ],
    )
    def embed(idx_hbm, table_hbm, out_hbm, lab_v, rows_v, sem):
        wid = lax.axis_index("s") * info.num_cores + lax.axis_index("c")
        base = wid * b_per_w
        pltpu.sync_copy(idx_hbm.at[pl.ds(base, b_per_w)], lab_v)

        def group_body(g, carry):
            off = g * 16
            labs = lab_v[pl.ds(off, 16)]
            t_vec = lax.shift_right_logical(labs, 3)
            s_vec = lax.bitwise_and(labs, 7)
            for k in range(16):
                t = t_vec[k]
                s = s_vec[k]
                pltpu.async_copy(table_hbm.at[t, s], rows_v.at[off + k], sem)
            return carry

        lax.fori_loop(0, b_per_w // 16, group_body, 0)
        pltpu.make_async_copy(
            out_hbm.at[pl.ds(base, b_per_w)], rows_v, sem
        ).wait()
        pltpu.sync_copy(rows_v, out_hbm.at[pl.ds(base, b_per_w)])

    return embed


def kernel(class_labels, table):
    B = class_labels.shape[0]
    V, D = table.shape
    embed = _build_embed_kernel(B, V, D)
    t3 = table.reshape(V // 8, 8, D)
    out = embed(class_labels.astype(jnp.int32), t3)
    return out[:, None, :]
